# baseline (speedup 1.0000x reference)
import jax
import jax.numpy as jnp
from jax import lax
from jax.experimental import pallas as pl
from jax.experimental.pallas import tpu as pltpu
from jax.experimental.pallas import tpu_sc as plsc

_NUM_TOKENS = 32768
_TOP_K = 8
_LANES = 16
_NUM_WORKERS = 32
_ROWS_PW = _NUM_TOKENS // _NUM_WORKERS
_VECS_PW = _ROWS_PW // 2


def _sc_body(idx_hbm, sc_hbm, out_s_hbm, out_i_hbm, idx_v, sc_v, os_v, oi_v):
    wid = lax.axis_index("s") * 2 + lax.axis_index("c")
    rbase = wid * _ROWS_PW
    pltpu.sync_copy(idx_hbm.at[pl.ds(rbase, _ROWS_PW)], idx_v)
    pltpu.sync_copy(sc_hbm.at[pl.ds(rbase, _ROWS_PW)], sc_v)

    lane = lax.iota(jnp.int32, _LANES)
    col = lane & 7
    half = lane >> 3
    bias = half << 9

    def body(j, carry):
        row = 2 * j + half
        keys = plsc.load_gather(idx_v, [row, col])
        scs = plsc.load_gather(sc_v, [row, col])
        comp = keys * 8 + col + bias
        sk, sv = plsc.sort_key_val(comp, scs)
        plsc.store_scatter(os_v, [row, col], sv)
        plsc.store_scatter(oi_v, [row, col], (sk & 7) >> 3)
        return carry

    lax.fori_loop(0, _VECS_PW, body, 0)

    pltpu.sync_copy(os_v, out_s_hbm.at[pl.ds(rbase, _ROWS_PW)])
    pltpu.sync_copy(oi_v, out_i_hbm.at[pl.ds(rbase, _ROWS_PW)])


def kernel(top_scores, selected_experts_indices):
    run = pl.kernel(
        _sc_body,
        out_type=(
            jax.ShapeDtypeStruct((_NUM_TOKENS, _TOP_K), jnp.float32),
            jax.ShapeDtypeStruct((_NUM_TOKENS, _TOP_K), jnp.int32),
        ),
        mesh=plsc.VectorSubcoreMesh(core_axis_name="c", subcore_axis_name="s"),
        compiler_params=pltpu.CompilerParams(
            needs_layout_passes=False,
            use_tc_tiling_on_sc=False,
            skip_device_barrier=True,
        ),
        scratch_types=[
            pltpu.VMEM((_ROWS_PW, _TOP_K), jnp.int32),
            pltpu.VMEM((_ROWS_PW, _TOP_K), jnp.float32),
            pltpu.VMEM((_ROWS_PW, _TOP_K), jnp.float32),
            pltpu.VMEM((_ROWS_PW, _TOP_K), jnp.int32),
        ],
    )
    return run(selected_experts_indices, top_scores)


# speedup vs baseline: 1.1875x; 1.1875x over previous
import jax
import jax.numpy as jnp
from jax import lax
from jax.experimental import pallas as pl
from jax.experimental.pallas import tpu as pltpu
from jax.experimental.pallas import tpu_sc as plsc

_NUM_TOKENS = 32768
_TOP_K = 8
_LANES = 16
_NUM_WORKERS = 32
_ROWS_PW = _NUM_TOKENS // _NUM_WORKERS
_CHUNK = 128
_NCHUNKS = _ROWS_PW // _CHUNK
_VECS_PC = _CHUNK // 2


def _sc_body(idx_hbm, sc_hbm, out_s_hbm, out_i_hbm, idx_v, sc_v, os_v, oi_v):
    wid = lax.axis_index("s") * 2 + lax.axis_index("c")

    lane = lax.iota(jnp.int32, _LANES)
    col = lane & 7
    half = lane >> 3
    bias = half << 9

    def chunk_body(c, carry):
        rbase = wid * _ROWS_PW + c * _CHUNK
        pltpu.sync_copy(idx_hbm.at[pl.ds(rbase, _CHUNK)], idx_v)
        pltpu.sync_copy(sc_hbm.at[pl.ds(rbase, _CHUNK)], sc_v)

        def body(j, inner):
            row = 2 * j + half
            keys = plsc.load_gather(idx_v, [row, col])
            scs = plsc.load_gather(sc_v, [row, col])
            comp = keys * 8 + col + bias
            sk, sv = plsc.sort_key_val(comp, scs)
            plsc.store_scatter(os_v, [row, col], sv)
            plsc.store_scatter(oi_v, [row, col], (sk & 7) >> 3)
            return inner

        lax.fori_loop(0, _VECS_PC, body, 0)

        pltpu.sync_copy(os_v, out_s_hbm.at[pl.ds(rbase, _CHUNK)])
        pltpu.sync_copy(oi_v, out_i_hbm.at[pl.ds(rbase, _CHUNK)])
        return carry

    lax.fori_loop(0, _NCHUNKS, chunk_body, 0)


def kernel(top_scores, selected_experts_indices):
    run = pl.kernel(
        _sc_body,
        out_type=(
            jax.ShapeDtypeStruct((_NUM_TOKENS, _TOP_K), jnp.float32),
            jax.ShapeDtypeStruct((_NUM_TOKENS, _TOP_K), jnp.int32),
        ),
        mesh=plsc.VectorSubcoreMesh(core_axis_name="c", subcore_axis_name="s"),
        compiler_params=pltpu.CompilerParams(
            needs_layout_passes=False,
            use_tc_tiling_on_sc=True,
            skip_device_barrier=True,
        ),
        scratch_types=[
            pltpu.VMEM((_CHUNK, _TOP_K), jnp.int32),
            pltpu.VMEM((_CHUNK, _TOP_K), jnp.float32),
            pltpu.VMEM((_CHUNK, _TOP_K), jnp.float32),
            pltpu.VMEM((_CHUNK, _TOP_K), jnp.int32),
        ],
    )
    return run(selected_experts_indices, top_scores)


# speedup vs baseline: 3.1200x; 2.6275x over previous
import jax
import jax.numpy as jnp
from jax import lax
from jax.experimental import pallas as pl
from jax.experimental.pallas import tpu as pltpu
from jax.experimental.pallas import tpu_sc as plsc

_NUM_TOKENS = 32768
_TOP_K = 8
_LANES = 16
_NUM_WORKERS = 32
_TOKS_PW = _NUM_TOKENS // _NUM_WORKERS
_VECS_PW = _TOKS_PW // 2


def _sc_body(idx_hbm, sc_hbm, out_s_hbm, out_i_hbm, idx_v, sc_v, os_v, oi_v):
    wid = lax.axis_index("s") * 2 + lax.axis_index("c")
    tbase = wid * _TOKS_PW
    pltpu.sync_copy(idx_hbm.at[:, pl.ds(tbase, _TOKS_PW)], idx_v)
    pltpu.sync_copy(sc_hbm.at[:, pl.ds(tbase, _TOKS_PW)], sc_v)

    lane = lax.iota(jnp.int32, _LANES)
    col = lane & 7
    half = lane >> 3
    bias = half << 9

    def body(j, carry):
        tok = 2 * j + half
        keys = plsc.load_gather(idx_v, [col, tok])
        scs = plsc.load_gather(sc_v, [col, tok])
        comp = keys * 8 + col + bias
        sk, sv = plsc.sort_key_val(comp, scs)
        plsc.store_scatter(os_v, [col, tok], sv)
        plsc.store_scatter(oi_v, [col, tok], (sk & 7) >> 3)
        return carry

    lax.fori_loop(0, _VECS_PW, body, 0)

    pltpu.sync_copy(os_v, out_s_hbm.at[:, pl.ds(tbase, _TOKS_PW)])
    pltpu.sync_copy(oi_v, out_i_hbm.at[:, pl.ds(tbase, _TOKS_PW)])


def kernel(top_scores, selected_experts_indices):
    run = pl.kernel(
        _sc_body,
        out_type=(
            jax.ShapeDtypeStruct((_TOP_K, _NUM_TOKENS), jnp.float32),
            jax.ShapeDtypeStruct((_TOP_K, _NUM_TOKENS), jnp.int32),
        ),
        mesh=plsc.VectorSubcoreMesh(core_axis_name="c", subcore_axis_name="s"),
        compiler_params=pltpu.CompilerParams(
            needs_layout_passes=False,
            use_tc_tiling_on_sc=True,
            skip_device_barrier=True,
        ),
        scratch_types=[
            pltpu.VMEM((_TOP_K, _TOKS_PW), jnp.int32),
            pltpu.VMEM((_TOP_K, _TOKS_PW), jnp.float32),
            pltpu.VMEM((_TOP_K, _TOKS_PW), jnp.float32),
            pltpu.VMEM((_TOP_K, _TOKS_PW), jnp.int32),
        ],
    )
    os_t, oi_t = run(selected_experts_indices.T, top_scores.T)
    return os_t.T, oi_t.T


# speedup vs baseline: 4.2774x; 1.3709x over previous
import jax
import jax.numpy as jnp
from jax import lax
from jax.experimental import pallas as pl
from jax.experimental.pallas import tpu as pltpu
from jax.experimental.pallas import tpu_sc as plsc

_NUM_TOKENS = 32768
_TOP_K = 8
_LANES = 16
_NUM_WORKERS = 32
_TOKS_PW = _NUM_TOKENS // _NUM_WORKERS
_VECS_PW = _TOKS_PW // 2


def _sc_body(idx_hbm, sc_hbm, out_s_hbm, out_i_hbm, idx_v, sc_v, os_v, oi_v):
    wid = lax.axis_index("s") * 2 + lax.axis_index("c")
    tbase = wid * _TOKS_PW
    pltpu.sync_copy(idx_hbm.at[:, pl.ds(tbase, _TOKS_PW)], idx_v)
    pltpu.sync_copy(sc_hbm.at[:, pl.ds(tbase, _TOKS_PW)], sc_v)

    lane = lax.iota(jnp.int32, _LANES)
    col = lane & 7
    half = lane >> 3
    bias = half << 9

    @plsc.parallel_loop(0, _VECS_PW, unroll=8)
    def body(j):
        tok = 2 * j + half
        keys = plsc.load_gather(idx_v, [col, tok])
        scs = plsc.load_gather(sc_v, [col, tok])
        comp = keys * 8 + col + bias
        sk, sv = plsc.sort_key_val(comp, scs)
        plsc.store_scatter(os_v, [col, tok], sv)
        plsc.store_scatter(oi_v, [col, tok], (sk & 7) >> 3)

    pltpu.sync_copy(os_v, out_s_hbm.at[:, pl.ds(tbase, _TOKS_PW)])
    pltpu.sync_copy(oi_v, out_i_hbm.at[:, pl.ds(tbase, _TOKS_PW)])


def kernel(top_scores, selected_experts_indices):
    run = pl.kernel(
        _sc_body,
        out_type=(
            jax.ShapeDtypeStruct((_TOP_K, _NUM_TOKENS), jnp.float32),
            jax.ShapeDtypeStruct((_TOP_K, _NUM_TOKENS), jnp.int32),
        ),
        mesh=plsc.VectorSubcoreMesh(core_axis_name="c", subcore_axis_name="s"),
        compiler_params=pltpu.CompilerParams(
            needs_layout_passes=False,
            use_tc_tiling_on_sc=True,
            skip_device_barrier=True,
        ),
        scratch_types=[
            pltpu.VMEM((_TOP_K, _TOKS_PW), jnp.int32),
            pltpu.VMEM((_TOP_K, _TOKS_PW), jnp.float32),
            pltpu.VMEM((_TOP_K, _TOKS_PW), jnp.float32),
            pltpu.VMEM((_TOP_K, _TOKS_PW), jnp.int32),
        ],
    )
    os_t, oi_t = run(selected_experts_indices.T, top_scores.T)
    return os_t.T, oi_t.T
